# baseline (speedup 1.0000x reference)
import jax
import jax.numpy as jnp
from jax import lax
from jax.experimental import pallas as pl
from jax.experimental.pallas import tpu as pltpu
from jax.experimental.pallas import tpu_sc as plsc

NUM_HEADS = 8
D_IN = 128
GREP = 128
HEAD_DIM = GREP // NUM_HEADS
NUM_SEGMENTS = 512
N_PAD = 102400
TC_BLK = 4096
SC_CHUNK = 128
CHUNKS_PER_TILE = N_PAD // 32 // SC_CHUNK
SEGS_PER_TILE = NUM_SEGMENTS // 16


def _leaky(x):
    return jnp.maximum(x, 0.01 * x)


def _mlp_body(nb, n_valid):
    def body(x_ref, b_ref, ws1, bs1, ws2, bs2, ws3e, bs3e,
             wt1, bt1, wt2, bt2, wt3, bt3, sel, vals_ref, den_ref):
        i = pl.program_id(0)

        @pl.when(i == 0)
        def _init():
            den_ref[:] = jnp.zeros_like(den_ref)

        x = x_ref[:]
        h = _leaky(jnp.dot(x, ws1[:], preferred_element_type=jnp.float32) + bs1[:])
        h = _leaky(jnp.dot(h, ws2[:], preferred_element_type=jnp.float32) + bs2[:])
        s = jnp.dot(h, ws3e[:], preferred_element_type=jnp.float32) + bs3e[:]

        t = _leaky(jnp.dot(x, wt1[:], preferred_element_type=jnp.float32) + bt1[:])
        t = _leaky(jnp.dot(t, wt2[:], preferred_element_type=jnp.float32) + bt2[:])
        r = _leaky(jnp.dot(t, wt3[:], preferred_element_type=jnp.float32) + bt3[:])

        ex = jnp.exp(s)
        row = (i * TC_BLK
               + jax.lax.broadcasted_iota(jnp.int32, (TC_BLK, 1), 0))
        valid = (row < n_valid).astype(jnp.float32)
        vals_ref[:] = ex * r * valid

        seg = b_ref[0, 0, :]
        onehot_t = (jax.lax.broadcasted_iota(jnp.int32, (NUM_SEGMENTS, TC_BLK), 0)
                    == seg[None, :]).astype(jnp.float32)
        ex8 = jnp.dot(ex * valid, sel[:], preferred_element_type=jnp.float32)
        den_ref[:] += jnp.dot(onehot_t, ex8, preferred_element_type=jnp.float32)

    return body


def _sc_scatter(vals, batch2d, zrows, out, vbuf, idx, acc):
    c = lax.axis_index("c")
    s = lax.axis_index("s")
    seg0 = s * SEGS_PER_TILE
    pltpu.sync_copy(zrows.at[pl.ds(seg0, SEGS_PER_TILE)],
                    acc.at[pl.ds(seg0, SEGS_PER_TILE)])
    plsc.subcore_barrier()
    base_chunk = (c * 16 + s) * CHUNKS_PER_TILE
    for j in range(CHUNKS_PER_TILE):
        g = base_chunk + j
        pltpu.sync_copy(vals.at[pl.ds(g * SC_CHUNK, SC_CHUNK)], vbuf)
        pltpu.sync_copy(batch2d.at[g], idx)
        pltpu.sync_copy(vbuf, acc.at[idx], add=True)
    plsc.subcore_barrier()
    pltpu.sync_copy(acc.at[pl.ds(seg0, SEGS_PER_TILE)],
                    out.at[c, pl.ds(seg0, SEGS_PER_TILE)])


def _combine_body(p0_ref, p1_ref, den_ref, expand, out_ref):
    num = p0_ref[:] + p1_ref[:]
    dexp = jnp.dot(den_ref[:], expand[:], preferred_element_type=jnp.float32)
    out_ref[:] = num / jnp.maximum(dexp, 1e-30)


def kernel(x, batch, Ws1, bs1, Ws2, bs2, Ws3, bs3, Wt1, bt1, Wt2, bt2, Wt3, bt3):
    n = x.shape[0]
    f32 = jnp.float32
    xp = jnp.concatenate([x, jnp.zeros((N_PAD - n, D_IN), f32)], axis=0)
    bp = jnp.concatenate([batch, jnp.zeros((N_PAD - n,), jnp.int32)], axis=0)
    nb = N_PAD // TC_BLK
    batch3 = bp.reshape(nb, 1, TC_BLK)
    batch2d = bp.reshape(N_PAD // SC_CHUNK, SC_CHUNK)

    ws3e = jnp.repeat(Ws3, HEAD_DIM, axis=1)
    bs3e = jnp.repeat(bs3, HEAD_DIM).reshape(1, GREP)
    sel = (jnp.arange(GREP)[:, None] == HEAD_DIM * jnp.arange(NUM_HEADS)[None, :]
           ).astype(f32)
    expand = (jnp.arange(GREP)[None, :] // HEAD_DIM == jnp.arange(NUM_HEADS)[:, None]
              ).astype(f32)
    b2 = lambda a: a.reshape(1, -1)

    full = lambda shape: pl.BlockSpec(shape, lambda i: (0, 0))
    vals, den = pl.pallas_call(
        _mlp_body(nb, n),
        grid=(nb,),
        in_specs=[
            pl.BlockSpec((TC_BLK, D_IN), lambda i: (i, 0)),
            pl.BlockSpec((1, 1, TC_BLK), lambda i: (i, 0, 0)),
            full((D_IN, 128)), full((1, 128)),
            full((128, 128)), full((1, 128)),
            full((128, GREP)), full((1, GREP)),
            full((D_IN, 128)), full((1, 128)),
            full((128, 128)), full((1, 128)),
            full((128, GREP)), full((1, GREP)),
            full((GREP, NUM_HEADS)),
        ],
        out_specs=[
            pl.BlockSpec((TC_BLK, GREP), lambda i: (i, 0)),
            pl.BlockSpec((NUM_SEGMENTS, NUM_HEADS), lambda i: (0, 0)),
        ],
        out_shape=[
            jax.ShapeDtypeStruct((N_PAD, GREP), f32),
            jax.ShapeDtypeStruct((NUM_SEGMENTS, NUM_HEADS), f32),
        ],
        compiler_params=pltpu.CompilerParams(
            dimension_semantics=("arbitrary",)),
    )(xp, batch3, Ws1, b2(bs1), Ws2, b2(bs2), ws3e, bs3e,
      Wt1, b2(bt1), Wt2, b2(bt2), Wt3, b2(bt3), sel)

    zrows = jnp.zeros((NUM_SEGMENTS, GREP), f32)
    mesh = plsc.VectorSubcoreMesh(core_axis_name="c", subcore_axis_name="s")
    partials = pl.kernel(
        _sc_scatter,
        out_type=jax.ShapeDtypeStruct((2, NUM_SEGMENTS, GREP), f32),
        mesh=mesh,
        scratch_types=[
            pltpu.VMEM((SC_CHUNK, GREP), f32),
            pltpu.VMEM((SC_CHUNK,), jnp.int32),
            pltpu.VMEM_SHARED((NUM_SEGMENTS, GREP), f32),
        ],
    )(vals, batch2d, zrows)

    return pl.pallas_call(
        _combine_body,
        grid=(1,),
        in_specs=[
            pl.BlockSpec((NUM_SEGMENTS, GREP), lambda i: (0, 0)),
            pl.BlockSpec((NUM_SEGMENTS, GREP), lambda i: (0, 0)),
            pl.BlockSpec((NUM_SEGMENTS, NUM_HEADS), lambda i: (0, 0)),
            full((NUM_HEADS, GREP)),
        ],
        out_specs=pl.BlockSpec((NUM_SEGMENTS, GREP), lambda i: (0, 0)),
        out_shape=jax.ShapeDtypeStruct((NUM_SEGMENTS, GREP), f32),
    )(partials[0], partials[1], den, expand)


# speedup vs baseline: 1.3319x; 1.3319x over previous
import jax
import jax.numpy as jnp
from jax.experimental import pallas as pl
from jax.experimental.pallas import tpu as pltpu

NUM_HEADS = 8
D_IN = 128
GREP = 128
HEAD_DIM = GREP // NUM_HEADS
NUM_SEGMENTS = 512


def _leaky(x):
    return jnp.maximum(x, 0.01 * x)


def _block_body(nb):
    def body(x_ref, b_ref, ws1, bs1, ws2, bs2, ws3e, bs3e,
             wt1, bt1, wt2, bt2, wt3, bt3, sel, expand,
             out_ref, den_ref):
        i = pl.program_id(0)

        @pl.when(i == 0)
        def _init():
            out_ref[:] = jnp.zeros_like(out_ref)
            den_ref[:] = jnp.zeros_like(den_ref)

        x = x_ref[:]
        h = _leaky(jnp.dot(x, ws1[:], preferred_element_type=jnp.float32) + bs1[:])
        h = _leaky(jnp.dot(h, ws2[:], preferred_element_type=jnp.float32) + bs2[:])
        s = jnp.dot(h, ws3e[:], preferred_element_type=jnp.float32) + bs3e[:]

        t = _leaky(jnp.dot(x, wt1[:], preferred_element_type=jnp.float32) + bt1[:])
        t = _leaky(jnp.dot(t, wt2[:], preferred_element_type=jnp.float32) + bt2[:])
        r = _leaky(jnp.dot(t, wt3[:], preferred_element_type=jnp.float32) + bt3[:])

        ex = jnp.exp(s)
        w = ex * r

        seg = b_ref[0, 0, :]
        blk = seg.shape[0]
        onehot_t = (jax.lax.broadcasted_iota(jnp.int32, (NUM_SEGMENTS, blk), 0)
                    == seg[None, :]).astype(jnp.float32)

        ex8 = jnp.dot(ex, sel[:], preferred_element_type=jnp.float32)
        rhs = jnp.concatenate([w, ex8], axis=1)
        upd = jnp.dot(onehot_t, rhs, preferred_element_type=jnp.float32)
        out_ref[:] += upd[:, :GREP]
        den_ref[:] += upd[:, GREP:]

        @pl.when(i == nb - 1)
        def _final():
            dexp = jnp.dot(den_ref[:], expand[:],
                           preferred_element_type=jnp.float32)
            out_ref[:] = out_ref[:] / jnp.maximum(dexp, 1e-30)

    return body


def kernel(x, batch, Ws1, bs1, Ws2, bs2, Ws3, bs3, Wt1, bt1, Wt2, bt2, Wt3, bt3):
    n = x.shape[0]
    blk = 5000 if n % 5000 == 0 else n
    nb = n // blk

    ws3e = jnp.repeat(Ws3, HEAD_DIM, axis=1)
    bs3e = jnp.repeat(bs3, HEAD_DIM).reshape(1, GREP)
    sel = (jnp.arange(GREP)[:, None] == HEAD_DIM * jnp.arange(NUM_HEADS)[None, :]
           ).astype(jnp.float32)
    expand = (jnp.arange(GREP)[None, :] // HEAD_DIM == jnp.arange(NUM_HEADS)[:, None]
              ).astype(jnp.float32)

    batch3 = batch.reshape(nb, 1, blk)
    b2 = lambda a: a.reshape(1, -1)

    full = lambda shape: pl.BlockSpec(shape, lambda i: (0, 0))
    return pl.pallas_call(
        _block_body(nb),
        grid=(nb,),
        in_specs=[
            pl.BlockSpec((blk, D_IN), lambda i: (i, 0)),
            pl.BlockSpec((1, 1, blk), lambda i: (i, 0, 0)),
            full((D_IN, 128)), full((1, 128)),
            full((128, 128)), full((1, 128)),
            full((128, GREP)), full((1, GREP)),
            full((D_IN, 128)), full((1, 128)),
            full((128, 128)), full((1, 128)),
            full((128, GREP)), full((1, GREP)),
            full((GREP, NUM_HEADS)), full((NUM_HEADS, GREP)),
        ],
        out_specs=pl.BlockSpec((NUM_SEGMENTS, GREP), lambda i: (0, 0)),
        out_shape=jax.ShapeDtypeStruct((NUM_SEGMENTS, GREP), jnp.float32),
        scratch_shapes=[pltpu.VMEM((NUM_SEGMENTS, NUM_HEADS), jnp.float32)],
        compiler_params=pltpu.CompilerParams(
            dimension_semantics=("arbitrary",)),
    )(x, batch3, Ws1, b2(bs1), Ws2, b2(bs2), ws3e, bs3e,
      Wt1, b2(bt1), Wt2, b2(bt2), Wt3, b2(bt3), sel, expand)


# speedup vs baseline: 1.6412x; 1.2323x over previous
import jax
import jax.numpy as jnp
from jax.experimental import pallas as pl
from jax.experimental.pallas import tpu as pltpu

NUM_HEADS = 8
D_IN = 128
GREP = 128
HEAD_DIM = GREP // NUM_HEADS
NUM_SEGMENTS = 512


def _leaky(x):
    return jnp.maximum(x, 0.01 * x)


def _block_body(nb):
    def body(x_ref, b_ref, ws1, bs1, ws2, bs2, ws3e, bs3e,
             wt1, bt1, wt2, bt2, wt3, bt3, sel, expand,
             out_ref, den_ref):
        i = pl.program_id(0)

        @pl.when(i == 0)
        def _init():
            out_ref[:] = jnp.zeros_like(out_ref)
            den_ref[:] = jnp.zeros_like(den_ref)

        x = x_ref[:]
        h = _leaky(jnp.dot(x, ws1[:], preferred_element_type=jnp.float32) + bs1[:])
        h = _leaky(jnp.dot(h, ws2[:], preferred_element_type=jnp.float32) + bs2[:])
        s = jnp.dot(h, ws3e[:], preferred_element_type=jnp.float32) + bs3e[:]

        t = _leaky(jnp.dot(x, wt1[:], preferred_element_type=jnp.float32) + bt1[:])
        t = _leaky(jnp.dot(t, wt2[:], preferred_element_type=jnp.float32) + bt2[:])
        r = _leaky(jnp.dot(t, wt3[:], preferred_element_type=jnp.float32) + bt3[:])

        ex = jnp.exp(s)
        w = ex * r

        seg = b_ref[0, 0, :]
        blk = seg.shape[0]
        onehot_t = (jax.lax.broadcasted_iota(jnp.int32, (NUM_SEGMENTS, blk), 0)
                    == seg[None, :]).astype(jnp.float32)

        ex8 = jnp.dot(ex, sel[:], preferred_element_type=jnp.float32)
        rhs = jnp.concatenate([w, ex8], axis=1)
        upd = jnp.dot(onehot_t, rhs, preferred_element_type=jnp.float32)
        out_ref[:] += upd[:, :GREP]
        den_ref[:] += upd[:, GREP:]

        @pl.when(i == nb - 1)
        def _final():
            dexp = jnp.dot(den_ref[:], expand[:],
                           preferred_element_type=jnp.float32)
            out_ref[:] = out_ref[:] / jnp.maximum(dexp, 1e-30)

    return body


def kernel(x, batch, Ws1, bs1, Ws2, bs2, Ws3, bs3, Wt1, bt1, Wt2, bt2, Wt3, bt3):
    n = x.shape[0]
    blk = 10000 if n % 10000 == 0 else n
    nb = n // blk

    ws3e = jnp.repeat(Ws3, HEAD_DIM, axis=1)
    bs3e = jnp.repeat(bs3, HEAD_DIM).reshape(1, GREP)
    sel = (jnp.arange(GREP)[:, None] == HEAD_DIM * jnp.arange(NUM_HEADS)[None, :]
           ).astype(jnp.float32)
    expand = (jnp.arange(GREP)[None, :] // HEAD_DIM == jnp.arange(NUM_HEADS)[:, None]
              ).astype(jnp.float32)

    batch3 = batch.reshape(nb, 1, blk)
    b2 = lambda a: a.reshape(1, -1)

    full = lambda shape: pl.BlockSpec(shape, lambda i: (0, 0))
    return pl.pallas_call(
        _block_body(nb),
        grid=(nb,),
        in_specs=[
            pl.BlockSpec((blk, D_IN), lambda i: (i, 0)),
            pl.BlockSpec((1, 1, blk), lambda i: (i, 0, 0)),
            full((D_IN, 128)), full((1, 128)),
            full((128, 128)), full((1, 128)),
            full((128, GREP)), full((1, GREP)),
            full((D_IN, 128)), full((1, 128)),
            full((128, 128)), full((1, 128)),
            full((128, GREP)), full((1, GREP)),
            full((GREP, NUM_HEADS)), full((NUM_HEADS, GREP)),
        ],
        out_specs=pl.BlockSpec((NUM_SEGMENTS, GREP), lambda i: (0, 0)),
        out_shape=jax.ShapeDtypeStruct((NUM_SEGMENTS, GREP), jnp.float32),
        scratch_shapes=[pltpu.VMEM((NUM_SEGMENTS, NUM_HEADS), jnp.float32)],
        compiler_params=pltpu.CompilerParams(
            dimension_semantics=("arbitrary",)),
    )(x, batch3, Ws1, b2(bs1), Ws2, b2(bs2), ws3e, bs3e,
      Wt1, b2(bt1), Wt2, b2(bt2), Wt3, b2(bt3), sel, expand)
